# baseline (speedup 1.0000x reference)
import jax
import jax.numpy as jnp
from jax import lax
from jax.experimental import pallas as pl
from jax.experimental.pallas import tpu as pltpu
from jax.experimental.pallas import tpu_sc as plsc

_NC, _NS = 2, 16
_NW = _NC * _NS
_SPAD = 64
_C = 128
_NBUF = 4


def kernel(x, weight):
    B, S = x.shape
    V, D = weight.shape
    n = B * _SPAD
    per_tile = n // _NW
    nchunks = per_tile // _C
    ngroups = nchunks // _NBUF
    xi = x.astype(jnp.int32)
    idx = jnp.concatenate([xi, xi[:, : _SPAD - S]], axis=1).reshape(n)

    mesh = plsc.VectorSubcoreMesh(core_axis_name="c", subcore_axis_name="s")

    @pl.kernel(
        out_type=jax.ShapeDtypeStruct((n, D), weight.dtype),
        mesh=mesh,
        scratch_types=[
            pltpu.VMEM((per_tile,), jnp.int32),
            pltpu.VMEM((_NBUF, _C, D), jnp.float32),
            pltpu.SemaphoreType.DMA,
            pltpu.SemaphoreType.DMA,
            pltpu.SemaphoreType.DMA,
            pltpu.SemaphoreType.DMA,
            pltpu.SemaphoreType.DMA,
            pltpu.SemaphoreType.DMA,
            pltpu.SemaphoreType.DMA,
            pltpu.SemaphoreType.DMA,
        ],
    )
    def k(w_hbm, i_hbm, o_hbm, idx_v, bufs, g0, g1, g2, g3, w0, w1, w2, w3):
        gsems = (g0, g1, g2, g3)
        wsems = (w0, w1, w2, w3)
        wid = lax.axis_index("s") * _NC + lax.axis_index("c")
        base = wid * per_tile
        pltpu.sync_copy(i_hbm.at[pl.ds(base, per_tile)], idx_v)

        def gather_copy(c, b):
            return pltpu.make_async_copy(
                w_hbm.at[idx_v.at[pl.ds(c * _C, _C)]], bufs.at[b], gsems[b])

        def write_copy(c, b):
            return pltpu.make_async_copy(
                bufs.at[b], o_hbm.at[pl.ds(base + c * _C, _C)], wsems[b])

        for b in range(_NBUF):
            gather_copy(b, b).start()

        @pl.loop(0, ngroups - 1)
        def _(g):
            for b in range(_NBUF):
                c = g * _NBUF + b
                gather_copy(c, b).wait()
                write_copy(c, b).start()
            for b in range(_NBUF):
                c = g * _NBUF + b
                write_copy(c, b).wait()
                gather_copy(c + _NBUF, b).start()

        gl = ngroups - 1
        for b in range(_NBUF):
            c = gl * _NBUF + b
            gather_copy(c, b).wait()
            write_copy(c, b).start()
        for b in range(_NBUF):
            write_copy(gl * _NBUF + b, b).wait()

    out = k(weight, idx).reshape(B, _SPAD, D)
    return out[:, :S, :]


# speedup vs baseline: 1.4360x; 1.4360x over previous
import jax
import jax.numpy as jnp
from jax import lax
from jax.experimental import pallas as pl
from jax.experimental.pallas import tpu as pltpu
from jax.experimental.pallas import tpu_sc as plsc

_NC, _NS = 2, 16
_NW = _NC * _NS
_SPAD = 56
_BPC = 4
_NBUF = 4


def kernel(x, weight):
    B, S = x.shape
    V, D = weight.shape
    n_out = B * _SPAD
    n_idx = B * S
    bat_per_tile = B // _NW
    idx_per_tile = bat_per_tile * _SPAD
    out_per_tile = bat_per_tile * _SPAD
    nchunks = bat_per_tile // _BPC
    ngroups = nchunks // _NBUF
    cw = _BPC * _SPAD
    xi = x.astype(jnp.int32)
    idx = jnp.concatenate([xi, xi[:, : _SPAD - S]], axis=1).reshape(B * _SPAD)

    mesh = plsc.VectorSubcoreMesh(core_axis_name="c", subcore_axis_name="s")

    @pl.kernel(
        out_type=jax.ShapeDtypeStruct((n_out, D), weight.dtype),
        mesh=mesh,
        scratch_types=[
            pltpu.VMEM((idx_per_tile,), jnp.int32),
            pltpu.VMEM((_NBUF, cw, D), jnp.float32),
            pltpu.SemaphoreType.DMA,
            pltpu.SemaphoreType.DMA,
            pltpu.SemaphoreType.DMA,
            pltpu.SemaphoreType.DMA,
            pltpu.SemaphoreType.DMA,
            pltpu.SemaphoreType.DMA,
            pltpu.SemaphoreType.DMA,
            pltpu.SemaphoreType.DMA,
        ],
    )
    def k(w_hbm, i_hbm, o_hbm, idx_v, bufs, g0, g1, g2, g3, w0, w1, w2, w3):
        gsems = (g0, g1, g2, g3)
        wsems = (w0, w1, w2, w3)
        wid = lax.axis_index("s") * _NC + lax.axis_index("c")
        ibase = wid * idx_per_tile
        obase = wid * out_per_tile
        pltpu.sync_copy(i_hbm.at[pl.ds(ibase, idx_per_tile)], idx_v)

        def gather_copies(c, b):
            return [
                pltpu.make_async_copy(
                    w_hbm.at[idx_v.at[pl.ds((c * _BPC + j) * _SPAD, S)]],
                    bufs.at[b].at[pl.ds(j * _SPAD, S)],
                    gsems[b])
                for j in range(_BPC)
            ]

        def write_copy(c, b):
            return pltpu.make_async_copy(
                bufs.at[b], o_hbm.at[pl.ds(obase + c * cw, cw)], wsems[b])

        for b in range(_NBUF):
            for cp in gather_copies(b, b):
                cp.start()

        @pl.loop(0, ngroups - 1)
        def _(g):
            for b in range(_NBUF):
                c = g * _NBUF + b
                for cp in gather_copies(c, b):
                    cp.wait()
                write_copy(c, b).start()
            for b in range(_NBUF):
                c = g * _NBUF + b
                write_copy(c, b).wait()
                for cp in gather_copies(c + _NBUF, b):
                    cp.start()

        gl = ngroups - 1
        for b in range(_NBUF):
            c = gl * _NBUF + b
            for cp in gather_copies(c, b):
                cp.wait()
            write_copy(c, b).start()
        for b in range(_NBUF):
            write_copy(gl * _NBUF + b, b).wait()

    out = k(weight, idx).reshape(B, _SPAD, D)
    return out[:, :S, :]


# speedup vs baseline: 2.8859x; 2.0097x over previous
import jax
import jax.numpy as jnp
from jax import lax
from jax.experimental import pallas as pl
from jax.experimental.pallas import tpu as pltpu
from jax.experimental.pallas import tpu_sc as plsc

_NC, _NS = 2, 16
_NW = _NC * _NS
_C = 200
_NBUF = 4


def kernel(x, weight):
    B, S = x.shape
    V, D = weight.shape
    n = B * S
    per_tile = n // _NW
    nchunks = per_tile // _C
    ngroups = nchunks // _NBUF
    idx = jnp.swapaxes(x, 0, 1).reshape(n).astype(jnp.int32)

    mesh = plsc.VectorSubcoreMesh(core_axis_name="c", subcore_axis_name="s")

    @pl.kernel(
        out_type=jax.ShapeDtypeStruct((n, D), weight.dtype),
        mesh=mesh,
        scratch_types=[
            pltpu.VMEM((per_tile,), jnp.int32),
            pltpu.VMEM((_NBUF, _C, D), jnp.float32),
            pltpu.SemaphoreType.DMA,
            pltpu.SemaphoreType.DMA,
            pltpu.SemaphoreType.DMA,
            pltpu.SemaphoreType.DMA,
            pltpu.SemaphoreType.DMA,
            pltpu.SemaphoreType.DMA,
            pltpu.SemaphoreType.DMA,
            pltpu.SemaphoreType.DMA,
        ],
    )
    def k(w_hbm, i_hbm, o_hbm, idx_v, bufs, g0, g1, g2, g3, w0, w1, w2, w3):
        gsems = (g0, g1, g2, g3)
        wsems = (w0, w1, w2, w3)
        wid = lax.axis_index("s") * _NC + lax.axis_index("c")
        base = wid * per_tile
        pltpu.sync_copy(i_hbm.at[pl.ds(base, per_tile)], idx_v)

        def gather_copy(c, b):
            return pltpu.make_async_copy(
                w_hbm.at[idx_v.at[pl.ds(c * _C, _C)]], bufs.at[b], gsems[b])

        def write_copy(c, b):
            return pltpu.make_async_copy(
                bufs.at[b], o_hbm.at[pl.ds(base + c * _C, _C)], wsems[b])

        for b in range(_NBUF):
            gather_copy(b, b).start()

        @pl.loop(0, ngroups - 1)
        def _(g):
            for b in range(_NBUF):
                c = g * _NBUF + b
                gather_copy(c, b).wait()
                write_copy(c, b).start()
            for b in range(_NBUF):
                c = g * _NBUF + b
                write_copy(c, b).wait()
                gather_copy(c + _NBUF, b).start()

        gl = ngroups - 1
        for b in range(_NBUF):
            c = gl * _NBUF + b
            gather_copy(c, b).wait()
            write_copy(c, b).start()
        for b in range(_NBUF):
            write_copy(gl * _NBUF + b, b).wait()

    out = k(weight, idx).reshape(S, B, D)
    return jnp.swapaxes(out, 0, 1)
